# baseline (speedup 1.0000x reference)
import jax
import jax.numpy as jnp
from jax import lax
from jax.experimental import pallas as pl
from jax.experimental.pallas import tpu as pltpu
from jax.experimental.pallas import tpu_sc as plsc

NUM_INPUT = 4096
HIDDEN_BATCHES = 23
HIDDEN_SIZE = 4096
NUM_OUTPUT = 1024
FAN_IN = 32
LANES = 16
NUM_TILES = 16
ROWS_HID = HIDDEN_SIZE // NUM_TILES
ROWS_OUT = NUM_OUTPUT // NUM_TILES


def _rows16(wbuf, ibuf, vals, bias_vec, row0, pstart):
    pvec = jnp.full((LANES,), pstart, dtype=jnp.int32)
    prods = []
    for i in range(LANES):
        r = row0 + i
        gi0 = ibuf[r, pl.ds(0, LANES)] - pvec
        gi1 = ibuf[r, pl.ds(LANES, LANES)] - pvec
        w0 = wbuf[r, pl.ds(0, LANES)]
        w1 = wbuf[r, pl.ds(LANES, LANES)]
        g0 = plsc.load_gather(vals, [gi0])
        g1 = plsc.load_gather(vals, [gi1])
        prods.append(w0 * g0 + w1 * g1)
    lane = lax.iota(jnp.int32, LANES)
    d = 1
    while len(prods) > 1:
        pidx = lane ^ d
        m = (lane & d) == 0
        nxt = []
        for k in range(0, len(prods), 2):
            a, b = prods[k], prods[k + 1]
            pa = jnp.take_along_axis(a, pidx, axis=0)
            pb = jnp.take_along_axis(b, pidx, axis=0)
            nxt.append(jnp.where(m, a, pb) + jnp.where(m, pa, b))
        prods = nxt
        d *= 2
    return prods[0] + bias_vec


def _body(x_hbm, hw_hbm, ow_hbm, bias_hbm, hi_hbm, oi_hbm, out_hbm,
          vals, wbuf, ibuf, owbuf, oibuf, bbuf, obuf, shared):
    cid = lax.axis_index("c")
    sid = lax.axis_index("s")

    @pl.when(cid == 0)
    def _():
        base = sid * ROWS_HID
        pltpu.sync_copy(x_hbm, vals)

        def layer(t, carry):
            pltpu.sync_copy(hw_hbm.at[t, pl.ds(base, ROWS_HID), :], wbuf)
            pltpu.sync_copy(hi_hbm.at[t, pl.ds(base, ROWS_HID), :], ibuf)
            pltpu.sync_copy(bias_hbm.at[pl.ds(t * HIDDEN_SIZE + base, ROWS_HID)], bbuf)
            pstart = t * HIDDEN_SIZE

            def rows(r, c2):
                row0 = r * LANES
                bv = bbuf[pl.ds(row0, LANES)]
                a = _rows16(wbuf, ibuf, vals, bv, row0, pstart)
                obuf[pl.ds(row0, LANES)] = a / (1.0 + jnp.exp(-a))
                return c2

            lax.fori_loop(0, ROWS_HID // LANES, rows, 0)

            slot = lax.rem(t, 2)
            pltpu.sync_copy(obuf, shared.at[slot, pl.ds(base, ROWS_HID)])
            plsc.subcore_barrier()
            pltpu.sync_copy(shared.at[slot], vals)
            return carry

        lax.fori_loop(0, HIDDEN_BATCHES, layer, 0)

        base_o = sid * ROWS_OUT
        pltpu.sync_copy(ow_hbm.at[pl.ds(base_o, ROWS_OUT), :], owbuf)
        pltpu.sync_copy(oi_hbm.at[pl.ds(base_o, ROWS_OUT), :], oibuf)
        pltpu.sync_copy(
            bias_hbm.at[pl.ds(HIDDEN_BATCHES * HIDDEN_SIZE + base_o, ROWS_OUT)],
            bbuf.at[pl.ds(0, ROWS_OUT)])
        pstart_o = HIDDEN_BATCHES * HIDDEN_SIZE

        def out_rows(r, c2):
            row0 = r * LANES
            bv = bbuf[pl.ds(row0, LANES)]
            obuf[pl.ds(row0, LANES)] = _rows16(owbuf, oibuf, vals, bv, row0, pstart_o)
            return c2

        lax.fori_loop(0, ROWS_OUT // LANES, out_rows, 0)
        pltpu.sync_copy(obuf.at[pl.ds(0, ROWS_OUT)], out_hbm.at[pl.ds(base_o, ROWS_OUT)])


def kernel(x, hidden_weights, out_weights, bias, hidden_idx, out_idx):
    mesh = plsc.VectorSubcoreMesh(core_axis_name="c", subcore_axis_name="s")
    run = pl.kernel(
        _body,
        mesh=mesh,
        compiler_params=pltpu.CompilerParams(
            use_tc_tiling_on_sc=False, needs_layout_passes=False),
        out_type=jax.ShapeDtypeStruct((NUM_OUTPUT,), jnp.float32),
        scratch_types=[
            pltpu.VMEM((HIDDEN_SIZE,), jnp.float32),
            pltpu.VMEM((ROWS_HID, FAN_IN), jnp.float32),
            pltpu.VMEM((ROWS_HID, FAN_IN), jnp.int32),
            pltpu.VMEM((ROWS_OUT, FAN_IN), jnp.float32),
            pltpu.VMEM((ROWS_OUT, FAN_IN), jnp.int32),
            pltpu.VMEM((ROWS_HID,), jnp.float32),
            pltpu.VMEM((ROWS_HID,), jnp.float32),
            pltpu.VMEM_SHARED((2, HIDDEN_SIZE), jnp.float32),
        ],
    )
    return run(x, hidden_weights, out_weights, bias, hidden_idx, out_idx)
